# baseline (device time: 55146 ns/iter reference)
import jax
import jax.numpy as jnp
from jax import lax
from jax.experimental import pallas as pl
from jax.experimental.pallas import tpu as pltpu

N_DEV = 32
N_FWD = N_DEV // 2
N_BWD = N_DEV - 1 - N_FWD

_PLANE = [(0, 0), (1, 0), (1, 1), (0, 1), (0, 2), (1, 2), (1, 3), (0, 3)]
_POS = {(x, y, z): 8 * z + i for z in range(4) for i, (x, y) in enumerate(_PLANE)}

_C16 = [(0, 0), (0, 1), (0, 2), (0, 3), (1, 3), (1, 2), (1, 1), (2, 1),
        (2, 2), (2, 3), (3, 3), (3, 2), (3, 1), (3, 0), (2, 0), (1, 0)]
_CYCLE_COORDS = [(0, y, z) for (y, z) in _C16] + \
                [(1, y, z) for (y, z) in reversed(_C16)]
PERM = [_POS[c] for c in _CYCLE_COORDS]
INV = [0] * N_DEV
for _j, _p in enumerate(PERM):
    INV[_p] = _j


def kernel(x, w_mat):
    m_per, k = x.shape
    _, n_per = w_mat.shape
    m_total = N_DEV * m_per

    def body(x_ref, w_ref, perm_ref, inv_ref, out_ref, comm_ref,
             send_f, recv_f, send_b, recv_b):
        my = lax.axis_index("i")
        cpos = inv_ref[my]

        def pos(cslot):
            return perm_ref[(cslot % N_DEV).astype(jnp.int32)]

        left = pos(cpos - 1)
        right = pos(cpos + 1)

        barrier_sem = pltpu.get_barrier_semaphore()
        for nbr in (left, right):
            pl.semaphore_signal(
                barrier_sem, inc=1,
                device_id=(nbr,), device_id_type=pl.DeviceIdType.MESH,
            )
        pl.semaphore_wait(barrier_sem, 2)

        comm_ref[pl.ds(my * m_per, m_per), :] = x_ref[...]

        def slot(o):
            return comm_ref.at[pl.ds(o * m_per, m_per), :]

        def xfer(q, ssem, rsem, target, from_own=False):
            o = pos(q)
            return pltpu.make_async_remote_copy(
                src_ref=x_ref if from_own else slot(o), dst_ref=slot(o),
                send_sem=ssem, recv_sem=rsem,
                device_id=(target,), device_id_type=pl.DeviceIdType.MESH,
            )

        def fwd(s):
            return xfer(cpos - s, send_f.at[s], recv_f.at[s], right,
                        from_own=(s == 0))

        def fwd_in(s):
            return xfer(cpos - 1 - s, send_f.at[s], recv_f.at[s], right)

        def bwd(s):
            return xfer(cpos + s, send_b.at[s], recv_b.at[s], left,
                        from_own=(s == 0))

        def bwd_in(s):
            return xfer(cpos + 1 + s, send_b.at[s], recv_b.at[s], left)

        started = []

        def start(r):
            r.start()
            started.append(r)

        start(fwd(0))
        start(bwd(0))

        for s in range(1, max(N_FWD, N_BWD)):
            if s < N_FWD:
                fwd_in(s - 1).wait_recv()
                start(fwd(s))
            if s < N_BWD:
                bwd_in(s - 1).wait_recv()
                start(bwd(s))

        fwd_in(N_FWD - 1).wait_recv()
        bwd_in(N_BWD - 1).wait_recv()

        for r in started:
            r.wait_send()

        y = jnp.dot(comm_ref[...], w_ref[...],
                    preferred_element_type=jnp.float32)
        out_ref[...] = jnp.maximum(y, 0.0)

    return pl.pallas_call(
        body,
        out_shape=jax.ShapeDtypeStruct((m_total, n_per), jnp.float32),
        in_specs=[
            pl.BlockSpec(memory_space=pltpu.VMEM),
            pl.BlockSpec(memory_space=pltpu.VMEM),
            pl.BlockSpec(memory_space=pltpu.SMEM),
            pl.BlockSpec(memory_space=pltpu.SMEM),
        ],
        out_specs=pl.BlockSpec(memory_space=pltpu.VMEM),
        scratch_shapes=[
            pltpu.VMEM((m_total, k), jnp.float32),
            pltpu.SemaphoreType.DMA((N_FWD,)),
            pltpu.SemaphoreType.DMA((N_FWD,)),
            pltpu.SemaphoreType.DMA((N_BWD,)),
            pltpu.SemaphoreType.DMA((N_BWD,)),
        ],
        compiler_params=pltpu.CompilerParams(collective_id=0),
    )(x, w_mat,
      jnp.asarray(PERM, dtype=jnp.int32),
      jnp.asarray(INV, dtype=jnp.int32))


# device time: 36486 ns/iter; 1.5114x vs baseline; 1.5114x over previous
import jax
import jax.numpy as jnp
from jax import lax
from jax.experimental import pallas as pl
from jax.experimental.pallas import tpu as pltpu

N_DEV = 32
HALF = N_DEV // 2
STEPS = 8

_PLANE = [(0, 0), (1, 0), (1, 1), (0, 1), (0, 2), (1, 2), (1, 3), (0, 3)]
_POS = {(x, y, z): 8 * z + i for z in range(4) for i, (x, y) in enumerate(_PLANE)}

_C16 = [(0, 0), (0, 1), (0, 2), (0, 3), (1, 3), (1, 2), (1, 1), (2, 1),
        (2, 2), (2, 3), (3, 3), (3, 2), (3, 1), (3, 0), (2, 0), (1, 0)]
_CYCLE_COORDS = [(0, y, z) for (y, z) in _C16] + \
                [(1, y, z) for (y, z) in _C16]
PERM = [_POS[c] for c in _CYCLE_COORDS]
INV = [0] * N_DEV
for _j, _p in enumerate(PERM):
    INV[_p] = _j


def kernel(x, w_mat):
    m_per, k = x.shape
    _, n_per = w_mat.shape
    m_total = N_DEV * m_per

    def body(x_ref, w_ref, perm_ref, inv_ref, out_ref, comm_ref,
             r1s, r1r, r2s, r2r, l1s, l1r, l2s, l2r, js, jr):
        my = lax.axis_index("i")
        cpos = inv_ref[my]

        def pos(cslot):
            return perm_ref[(cslot % N_DEV).astype(jnp.int32)]

        left = pos(cpos - 1)
        right = pos(cpos + 1)
        opp = pos(cpos + HALF)

        barrier_sem = pltpu.get_barrier_semaphore()
        for nbr in (left, right, opp):
            pl.semaphore_signal(
                barrier_sem, inc=1,
                device_id=(nbr,), device_id_type=pl.DeviceIdType.MESH,
            )
        pl.semaphore_wait(barrier_sem, 3)

        comm_ref[pl.ds(my * m_per, m_per), :] = x_ref[...]

        def slot(o):
            return comm_ref.at[pl.ds(o * m_per, m_per), :]

        def xfer(q, ssem, rsem, target, from_own=False):
            o = pos(q)
            return pltpu.make_async_remote_copy(
                src_ref=x_ref if from_own else slot(o), dst_ref=slot(o),
                send_sem=ssem, recv_sem=rsem,
                device_id=(target,), device_id_type=pl.DeviceIdType.MESH,
            )

        def R1(s):
            return xfer(cpos - s, r1s.at[s], r1r.at[s], right,
                        from_own=(s == 0))

        def L1(s):
            return xfer(cpos + s, l1s.at[s], l1r.at[s], left,
                        from_own=(s == 0))

        def R2(s):
            return xfer(cpos + 17 - s, r2s.at[s], r2r.at[s], right)

        def L2(s):
            return xfer(cpos - 17 + s, l2s.at[s], l2r.at[s], left)

        def R1_in(s):
            return xfer(cpos - 1 - s, r1s.at[s], r1r.at[s], right)

        def L1_in(s):
            return xfer(cpos + 1 + s, l1s.at[s], l1r.at[s], left)

        def R2_in(s):
            return xfer(cpos + 16 - s, r2s.at[s], r2r.at[s], right)

        def L2_in(s):
            return xfer(cpos - 16 + s, l2s.at[s], l2r.at[s], left)

        started = []

        def start(r):
            r.start()
            started.append(r)

        start(xfer(cpos, js.at[0], jr.at[0], opp, from_own=True))
        jump_in = xfer(cpos + HALF, js.at[0], jr.at[0], opp)

        start(R1(0))
        start(L1(0))

        for s in range(1, STEPS):
            R1_in(s - 1).wait_recv()
            start(R1(s))
            L1_in(s - 1).wait_recv()
            start(L1(s))
            if s == 1:
                jump_in.wait_recv()
            else:
                R2_in(s - 1).wait_recv()
                L2_in(s - 1).wait_recv()
            start(R2(s))
            start(L2(s))

        R1_in(STEPS - 1).wait_recv()
        L1_in(STEPS - 1).wait_recv()
        R2_in(STEPS - 1).wait_recv()
        L2_in(STEPS - 1).wait_recv()

        for r in started:
            r.wait_send()

        y = jnp.dot(comm_ref[...], w_ref[...],
                    preferred_element_type=jnp.float32)
        out_ref[...] = jnp.maximum(y, 0.0)

    return pl.pallas_call(
        body,
        out_shape=jax.ShapeDtypeStruct((m_total, n_per), jnp.float32),
        in_specs=[
            pl.BlockSpec(memory_space=pltpu.VMEM),
            pl.BlockSpec(memory_space=pltpu.VMEM),
            pl.BlockSpec(memory_space=pltpu.SMEM),
            pl.BlockSpec(memory_space=pltpu.SMEM),
        ],
        out_specs=pl.BlockSpec(memory_space=pltpu.VMEM),
        scratch_shapes=[
            pltpu.VMEM((m_total, k), jnp.float32),
            pltpu.SemaphoreType.DMA((STEPS,)),
            pltpu.SemaphoreType.DMA((STEPS,)),
            pltpu.SemaphoreType.DMA((STEPS,)),
            pltpu.SemaphoreType.DMA((STEPS,)),
            pltpu.SemaphoreType.DMA((STEPS,)),
            pltpu.SemaphoreType.DMA((STEPS,)),
            pltpu.SemaphoreType.DMA((STEPS,)),
            pltpu.SemaphoreType.DMA((STEPS,)),
            pltpu.SemaphoreType.DMA((1,)),
            pltpu.SemaphoreType.DMA((1,)),
        ],
        compiler_params=pltpu.CompilerParams(collective_id=0),
    )(x, w_mat,
      jnp.asarray(PERM, dtype=jnp.int32),
      jnp.asarray(INV, dtype=jnp.int32))
